# Initial kernel scaffold; baseline (speedup 1.0000x reference)
import jax
import jax.numpy as jnp
from jax.experimental import pallas as pl


def kernel(goal, cnt_table, val_table, W, b):
    raise NotImplementedError("write your pallas kernel here")


# speedup vs baseline: 1.0483x; 1.0483x over previous
import functools

import jax
import jax.numpy as jnp
from jax import lax
from jax.experimental import pallas as pl
from jax.experimental.pallas import tpu as pltpu
from jax.experimental.pallas import tpu_sc as plsc

B, K, NE, NH = 16384, 26, 32, 128
NC, NS, L = 2, 16, 16
NW = NC * NS
ROWS_W = B // NW
CHUNK = 4
NIDX = CHUNK * K
NSTEP = ROWS_W // CHUNK


def _sc_gather_mul(cnt_table, val_table, cnt_idx, val_idx):
    mesh = plsc.VectorSubcoreMesh(core_axis_name="c", subcore_axis_name="s")

    @functools.partial(
        pl.kernel,
        mesh=mesh,
        out_type=jax.ShapeDtypeStruct((B * K, NE), jnp.float32),
        scratch_types=[
            pltpu.VMEM((ROWS_W * K,), jnp.int32),
            pltpu.VMEM((ROWS_W * K,), jnp.int32),
            pltpu.VMEM((NIDX, NE), jnp.float32),
            pltpu.VMEM((NIDX, NE), jnp.float32),
            pltpu.VMEM((NIDX, NE), jnp.float32),
            pltpu.SemaphoreType.DMA,
            pltpu.SemaphoreType.DMA,
        ],
        compiler_params=pltpu.CompilerParams(use_tc_tiling_on_sc=False),
    )
    def k(cnt_hbm, val_hbm, cidx_hbm, vidx_hbm, out_hbm,
          cidx_v, vidx_v, crow_v, vrow_v, hrow_v, gsem, gsem2):
        wid = lax.axis_index("s") * NC + lax.axis_index("c")
        ibase = wid * (ROWS_W * K)
        pltpu.sync_copy(cidx_hbm.at[pl.ds(ibase, ROWS_W * K)], cidx_v)
        pltpu.sync_copy(vidx_hbm.at[pl.ds(ibase, ROWS_W * K)], vidx_v)

        def step(g, _):
            off = g * NIDX
            c1 = pltpu.async_copy(
                cnt_hbm.at[cidx_v.at[pl.ds(off, NIDX)]], crow_v, gsem)
            c2 = pltpu.async_copy(
                val_hbm.at[vidx_v.at[pl.ds(off, NIDX)]], vrow_v, gsem2)
            c1.wait()
            c2.wait()

            def mul(j, _):
                for e in range(0, NE, L):
                    hrow_v[j, pl.ds(e, L)] = (
                        crow_v[j, pl.ds(e, L)] * vrow_v[j, pl.ds(e, L)])
                return 0

            lax.fori_loop(0, NIDX, mul, 0)
            pltpu.sync_copy(hrow_v, out_hbm.at[pl.ds(ibase + off, NIDX)])
            return 0

        lax.fori_loop(0, NSTEP, step, 0)

    return k(cnt_table, val_table, cnt_idx, val_idx)


def _tc_body(h_ref, w_ref, b_ref, o_ref):
    h = jnp.tanh(h_ref[...])
    o_ref[...] = (
        jnp.dot(h, w_ref[...], preferred_element_type=jnp.float32)
        + b_ref[...])


def _tc_tanh_matmul(h, W, b):
    BM = 1024
    return pl.pallas_call(
        _tc_body,
        grid=(B // BM,),
        in_specs=[
            pl.BlockSpec((BM, K * NE), lambda i: (i, 0)),
            pl.BlockSpec((K * NE, NH), lambda i: (0, 0)),
            pl.BlockSpec((1, NH), lambda i: (0, 0)),
        ],
        out_specs=pl.BlockSpec((BM, NH), lambda i: (i, 0)),
        out_shape=jax.ShapeDtypeStruct((B, NH), jnp.float32),
    )(h, W, b)


@jax.jit
def kernel(goal, cnt_table, val_table, W, b):
    gi = goal.reshape(B, K, 2)
    cnt_idx = gi[:, :, 0].reshape(-1)
    val_idx = gi[:, :, 1].reshape(-1)
    h = _sc_gather_mul(cnt_table, val_table, cnt_idx, val_idx)
    h = h.reshape(B, K * NE)
    return _tc_tanh_matmul(h, W, b.reshape(1, NH))


# speedup vs baseline: 1.1074x; 1.0564x over previous
import functools

import jax
import jax.numpy as jnp
from jax import lax
from jax.experimental import pallas as pl
from jax.experimental.pallas import tpu as pltpu
from jax.experimental.pallas import tpu_sc as plsc

B, K, NE, NH = 16384, 26, 32, 128
NC, NS, L = 2, 16, 16
NW = NC * NS
ROWS_W = B // NW
CHUNK = 4
NIDX = CHUNK * K
NSTEP = ROWS_W // CHUNK
NBUF = 4
NOUTER = NSTEP // NBUF


def _sc_gather_mul(cnt_table, val_table, cnt_idx, val_idx):
    mesh = plsc.VectorSubcoreMesh(core_axis_name="c", subcore_axis_name="s")

    @functools.partial(
        pl.kernel,
        mesh=mesh,
        out_type=jax.ShapeDtypeStruct((B * K, NE), jnp.float32),
        scratch_types=[
            pltpu.VMEM((ROWS_W * K,), jnp.int32),
            pltpu.VMEM((ROWS_W * K,), jnp.int32),
            pltpu.VMEM((NBUF, NIDX, NE), jnp.float32),
            pltpu.VMEM((NBUF, NIDX, NE), jnp.float32),
            pltpu.VMEM((NBUF, NIDX, NE), jnp.float32),
            [pltpu.SemaphoreType.DMA for _ in range(NBUF)],
            [pltpu.SemaphoreType.DMA for _ in range(NBUF)],
            [pltpu.SemaphoreType.DMA for _ in range(NBUF)],
        ],
        compiler_params=pltpu.CompilerParams(use_tc_tiling_on_sc=False),
    )
    def k(cnt_hbm, val_hbm, cidx_hbm, vidx_hbm, out_hbm,
          cidx_v, vidx_v, crow_v, vrow_v, hrow_v, csems, vsems, osems):
        wid = lax.axis_index("s") * NC + lax.axis_index("c")
        ibase = wid * (ROWS_W * K)
        pltpu.sync_copy(cidx_hbm.at[pl.ds(ibase, ROWS_W * K)], cidx_v)
        pltpu.sync_copy(vidx_hbm.at[pl.ds(ibase, ROWS_W * K)], vidx_v)

        def start_gather(b, off):
            pltpu.async_copy(
                cnt_hbm.at[cidx_v.at[pl.ds(off, NIDX)]], crow_v.at[b],
                csems[b])
            pltpu.async_copy(
                val_hbm.at[vidx_v.at[pl.ds(off, NIDX)]], vrow_v.at[b],
                vsems[b])

        def wait_gather(b, off):
            pltpu.make_async_copy(
                cnt_hbm.at[cidx_v.at[pl.ds(off, NIDX)]], crow_v.at[b],
                csems[b]).wait()
            pltpu.make_async_copy(
                val_hbm.at[vidx_v.at[pl.ds(off, NIDX)]], vrow_v.at[b],
                vsems[b]).wait()

        for b in range(NBUF):
            start_gather(b, b * NIDX)

        def outer(o, _):
            for b in range(NBUF):
                off = (o * NBUF + b) * NIDX
                wait_gather(b, off)

                @pl.when(o >= 1)
                def _():
                    pltpu.make_async_copy(
                        hrow_v.at[b],
                        out_hbm.at[pl.ds(ibase + off - NBUF * NIDX, NIDX)],
                        osems[b]).wait()

                def mul(j, _):
                    for e in range(0, NE, L):
                        hrow_v[b, j, pl.ds(e, L)] = (
                            crow_v[b, j, pl.ds(e, L)]
                            * vrow_v[b, j, pl.ds(e, L)])
                    return 0

                lax.fori_loop(0, NIDX, mul, 0, unroll=4)
                pltpu.async_copy(
                    hrow_v.at[b], out_hbm.at[pl.ds(ibase + off, NIDX)],
                    osems[b])

                @pl.when(o < NOUTER - 1)
                def _():
                    start_gather(b, off + NBUF * NIDX)
            return 0

        lax.fori_loop(0, NOUTER, outer, 0)
        for b in range(NBUF):
            off = (NSTEP - NBUF + b) * NIDX
            pltpu.make_async_copy(
                hrow_v.at[b], out_hbm.at[pl.ds(ibase + off, NIDX)],
                osems[b]).wait()

    return k(cnt_table, val_table, cnt_idx, val_idx)


def _tc_body(h_ref, w_ref, b_ref, o_ref):
    h = jnp.tanh(h_ref[...])
    o_ref[...] = (
        jnp.dot(h, w_ref[...], preferred_element_type=jnp.float32)
        + b_ref[...])


def _tc_tanh_matmul(h, W, b):
    BM = 1024
    return pl.pallas_call(
        _tc_body,
        grid=(B // BM,),
        in_specs=[
            pl.BlockSpec((BM, K * NE), lambda i: (i, 0)),
            pl.BlockSpec((K * NE, NH), lambda i: (0, 0)),
            pl.BlockSpec((1, NH), lambda i: (0, 0)),
        ],
        out_specs=pl.BlockSpec((BM, NH), lambda i: (i, 0)),
        out_shape=jax.ShapeDtypeStruct((B, NH), jnp.float32),
    )(h, W, b)


@jax.jit
def kernel(goal, cnt_table, val_table, W, b):
    gi = goal.reshape(B, K, 2)
    cnt_idx = gi[:, :, 0].reshape(-1)
    val_idx = gi[:, :, 1].reshape(-1)
    h = _sc_gather_mul(cnt_table, val_table, cnt_idx, val_idx)
    h = h.reshape(B, K * NE)
    return _tc_tanh_matmul(h, W, b.reshape(1, NH))
